# Initial kernel scaffold; baseline (speedup 1.0000x reference)
import jax
import jax.numpy as jnp
from jax.experimental import pallas as pl


def kernel(a, b, coeffs, scale):
    raise NotImplementedError("write your pallas kernel here")


# speedup vs baseline: 307.4090x; 307.4090x over previous
import functools

import jax
import jax.numpy as jnp
from jax import lax
from jax.experimental import pallas as pl
from jax.experimental.pallas import tpu as pltpu
from jax.experimental.pallas import tpu_sc as plsc

_N = 1048576
_GRID = 16
_NC = 2
_NS = 16
_L = 16
_NW = _NC * _NS
_CPW = _N // _NW
_TBL = _GRID * _GRID * 3


def _tile_body(a_hbm, b_hbm, coeffs_hbm, scale_hbm, out_hbm,
               tbl_v, sc_v, a_v, b_v, o_v):
    wid = lax.axis_index("s") * _NC + lax.axis_index("c")
    base = wid * _CPW

    pltpu.sync_copy(coeffs_hbm, tbl_v)
    pltpu.sync_copy(scale_hbm, sc_v)
    sv = sc_v[...]

    def prep(i, carry):
        c = tbl_v[pl.ds(i * _L, _L)]
        q = jnp.where(c > 0.3, 1.0, jnp.where(c < -0.3, -1.0, 0.0))
        tbl_v[pl.ds(i * _L, _L)] = (c + (q - c)) * sv
        return carry

    lax.fori_loop(0, _TBL // _L, prep, 0)

    pltpu.sync_copy(a_hbm.at[pl.ds(base, _CPW)], a_v)
    pltpu.sync_copy(b_hbm.at[pl.ds(base, _CPW)], b_v)

    def body(vi, carry):
        off = vi * _L
        av = a_v[pl.ds(off, _L)]
        bv = b_v[pl.ds(off, _L)]
        xa = (av + 1.0) * 8.0
        xb = (bv + 1.0) * 8.0
        ia = jnp.minimum(jnp.maximum(xa.astype(jnp.int32), 0), _GRID - 1)
        ib = jnp.minimum(jnp.maximum(xb.astype(jnp.int32), 0), _GRID - 1)
        fl = ia * 48 + ib * 3
        q0 = plsc.load_gather(tbl_v, [fl])
        q1 = plsc.load_gather(tbl_v, [fl + 1])
        q2 = plsc.load_gather(tbl_v, [fl + 2])
        la = xa - ia.astype(jnp.float32)
        lb = xb - ib.astype(jnp.float32)
        o_v[pl.ds(off, _L)] = q0 + q1 * la + q2 * lb
        return carry

    lax.fori_loop(0, _CPW // _L, body, 0)

    pltpu.sync_copy(o_v, out_hbm.at[pl.ds(base, _CPW)])


def kernel(a, b, coeffs, scale):
    mesh = plsc.VectorSubcoreMesh(core_axis_name="c", subcore_axis_name="s")
    run = functools.partial(
        pl.kernel,
        mesh=mesh,
        compiler_params=pltpu.CompilerParams(needs_layout_passes=False),
        out_type=jax.ShapeDtypeStruct((_N,), jnp.float32),
        scratch_types=[
            pltpu.VMEM((_TBL,), jnp.float32),
            pltpu.VMEM((_L,), jnp.float32),
            pltpu.VMEM((_CPW,), jnp.float32),
            pltpu.VMEM((_CPW,), jnp.float32),
            pltpu.VMEM((_CPW,), jnp.float32),
        ],
    )(_tile_body)
    coeffs_flat = coeffs.reshape(_TBL)
    scale_vec = jnp.broadcast_to(scale, (_L,))
    return run(a, b, coeffs_flat, scale_vec)


# speedup vs baseline: 513.3323x; 1.6699x over previous
import functools

import jax
import jax.numpy as jnp
from jax import lax
from jax.experimental import pallas as pl
from jax.experimental.pallas import tpu as pltpu
from jax.experimental.pallas import tpu_sc as plsc

_N = 1048576
_GRID = 16
_NC = 2
_NS = 16
_L = 16
_NW = _NC * _NS
_CPW = _N // _NW
_TBL = _GRID * _GRID * 3


def _tile_body(a_hbm, b_hbm, coeffs_hbm, scale_hbm, out_hbm,
               tbl_v, sc_v, a_v, b_v, o_v):
    wid = lax.axis_index("s") * _NC + lax.axis_index("c")
    base = wid * _CPW

    pltpu.sync_copy(coeffs_hbm, tbl_v)
    pltpu.sync_copy(scale_hbm, sc_v)
    sv = sc_v[...]

    def prep(i, carry):
        c = tbl_v[pl.ds(i * _L, _L)]
        q = jnp.where(c > 0.3, 1.0, jnp.where(c < -0.3, -1.0, 0.0))
        tbl_v[pl.ds(i * _L, _L)] = (c + (q - c)) * sv
        return carry

    lax.fori_loop(0, _TBL // _L, prep, 0)

    pltpu.sync_copy(a_hbm.at[pl.ds(base, _CPW)], a_v)
    pltpu.sync_copy(b_hbm.at[pl.ds(base, _CPW)], b_v)

    @functools.partial(plsc.parallel_loop, 0, _CPW // _L, unroll=8)
    def body(vi):
        off = vi * _L
        av = a_v[pl.ds(off, _L)]
        bv = b_v[pl.ds(off, _L)]
        xa = (av + 1.0) * 8.0
        xb = (bv + 1.0) * 8.0
        ia = jnp.minimum(jnp.maximum(xa.astype(jnp.int32), 0), _GRID - 1)
        ib = jnp.minimum(jnp.maximum(xb.astype(jnp.int32), 0), _GRID - 1)
        fl = ia * 48 + ib * 3
        q0 = plsc.load_gather(tbl_v, [fl])
        q1 = plsc.load_gather(tbl_v, [fl + 1])
        q2 = plsc.load_gather(tbl_v, [fl + 2])
        la = xa - ia.astype(jnp.float32)
        lb = xb - ib.astype(jnp.float32)
        o_v[pl.ds(off, _L)] = q0 + q1 * la + q2 * lb

    pltpu.sync_copy(o_v, out_hbm.at[pl.ds(base, _CPW)])


def kernel(a, b, coeffs, scale):
    mesh = plsc.VectorSubcoreMesh(core_axis_name="c", subcore_axis_name="s")
    run = functools.partial(
        pl.kernel,
        mesh=mesh,
        compiler_params=pltpu.CompilerParams(needs_layout_passes=False),
        out_type=jax.ShapeDtypeStruct((_N,), jnp.float32),
        scratch_types=[
            pltpu.VMEM((_TBL,), jnp.float32),
            pltpu.VMEM((_L,), jnp.float32),
            pltpu.VMEM((_CPW,), jnp.float32),
            pltpu.VMEM((_CPW,), jnp.float32),
            pltpu.VMEM((_CPW,), jnp.float32),
        ],
    )(_tile_body)
    coeffs_flat = coeffs.reshape(_TBL)
    scale_vec = jnp.broadcast_to(scale, (_L,))
    return run(a, b, coeffs_flat, scale_vec)
